# baseline (speedup 1.0000x reference)
import functools

import jax
import jax.numpy as jnp
import numpy as np
from jax import lax
from jax.experimental import pallas as pl
from jax.experimental.pallas import tpu as pltpu
from jax.experimental.pallas import tpu_sc as plsc

_FIELD = 40000
_NF = 26
_NE = 22
_B = 16384
_D = 16
_NW = 32
_PT = _B // _NW
_C = 64
_G = _PT // _C
_KG = _C * _NE // 16
_WR = _C * _NE * _D // 128
_LR = _C * _NF // 128
_OFFS = np.arange(_NF, dtype=np.int32) * _FIELD

_mesh = plsc.VectorSubcoreMesh(core_axis_name="c", subcore_axis_name="s")


@functools.partial(
    pl.kernel,
    out_type=jax.ShapeDtypeStruct((_B,), jnp.float32),
    mesh=_mesh,
    compiler_params=pltpu.CompilerParams(
        needs_layout_passes=False, use_tc_tiling_on_sc=False),
    scratch_types=[
        pltpu.VMEM((_C * _NE,), jnp.int32),
        pltpu.VMEM((_C * _NF,), jnp.int32),
        pltpu.VMEM((_C * _NE * _D,), jnp.int32),
        pltpu.VMEM((_C * _NE * _D,), jnp.float32),
        pltpu.VMEM((_C * _NF + 16,), jnp.float32),
        pltpu.VMEM((_D * 16,), jnp.float32),
        pltpu.VMEM((_C,), jnp.float32),
        pltpu.VMEM((16,), jnp.float32),
        pltpu.SemaphoreType.DMA,
    ],
)
def _fm_sc(eidx_h, lidx_h, emb_h, lin_h, bias_h, out_h,
           eidx, lidx, widx, embbuf, linbuf, tbuf, zbuf, biasv, sem):
    wid = lax.axis_index("s") * 2 + lax.axis_index("c")
    pltpu.sync_copy(bias_h, biasv)
    lanes = lax.iota(jnp.int32, 16)
    i16 = lanes * 16
    mask10 = lanes < 10

    def chunk(g, carry):
        ch = wid * _G + g
        pltpu.sync_copy(eidx_h.at[pl.ds(ch * (_C * _NE), _C * _NE)], eidx)
        pltpu.sync_copy(lidx_h.at[pl.ds(ch * (_C * _NF), _C * _NF)], lidx)

        def expand(kg, c1):
            base16 = eidx[pl.ds(kg * 16, 16)] * 16
            off = kg * 256
            for d in range(_D):
                plsc.store_scatter(widx, [i16 + (off + d)], base16 + d)
            return c1

        lax.fori_loop(0, _KG, expand, 0)

        lcps = []
        for j in range(_LR):
            lcps.append(pltpu.async_copy(
                lin_h.at[lidx.at[pl.ds(j * 128, 128)]],
                linbuf.at[pl.ds(j * 128, 128)], sem))

        def gfire(q, c1):
            cps = []
            for j in range(16):
                o = q * 2048 + j * 128
                cps.append(pltpu.async_copy(
                    emb_h.at[widx.at[pl.ds(o, 128)]],
                    embbuf.at[pl.ds(o, 128)], sem))
            for c in cps:
                c.wait()
            return c1

        lax.fori_loop(0, _WR // 16, gfire, 0)
        for c in lcps:
            c.wait()
        bv = biasv[...]

        def per_grp(grp, c2):
            def per_b(j, c3):
                b = grp * 16 + j
                eb = b * (_NE * _D)
                E = [embbuf[pl.ds(eb + i * _D, 16)] for i in range(_NE)]
                s = E[4]
                for i in range(5, 11):
                    s = s + E[i]
                acc = E[3] * s
                t = s + E[11] + E[12]
                t = t + E[3]
                acc = acc + E[2] * t
                t = t + E[2]
                acc = acc + E[1] * t
                t = t + E[1]
                acc = acc + E[0] * t
                for i in range(20):
                    acc = acc + E[i] * (E[i + 1] * E[i + 2])
                lb = b * _NF
                v1 = linbuf[pl.ds(lb, 16)]
                v2 = linbuf[pl.ds(lb + 16, 16)]
                v2 = jnp.where(mask10, v2, jnp.float32(0.0))
                acc = acc + v1 + v2 + bv
                plsc.store_scatter(tbuf, [i16 + j], acc)
                return c3

            lax.fori_loop(0, 16, per_b, 0)
            z = tbuf[pl.ds(0, 16)]
            for d in range(1, _D):
                z = z + tbuf[pl.ds(d * 16, 16)]
            zbuf[pl.ds(grp * 16, 16)] = 1.0 / (1.0 + jnp.exp(-z))
            return c2

        lax.fori_loop(0, _C // 16, per_grp, 0)
        pltpu.sync_copy(zbuf, out_h.at[pl.ds(ch * _C, _C)])
        return carry

    lax.fori_loop(0, _G, chunk, 0)


def kernel(x, emb_table, lin_table, lin_bias, genotype_2nd, genotype_3rd):
    del genotype_2nd, genotype_3rd
    xo = x + jnp.asarray(_OFFS)[None, :]
    eidx = xo[:, :_NE].reshape(-1)
    lidx = xo.reshape(-1)
    bias16 = jnp.pad(lin_bias.astype(jnp.float32), (0, 15))
    return _fm_sc(eidx, lidx, emb_table.reshape(-1), lin_table.reshape(-1),
                  bias16)


# speedup vs baseline: 1.5914x; 1.5914x over previous
import functools

import jax
import jax.numpy as jnp
import numpy as np
from jax import lax
from jax.experimental import pallas as pl
from jax.experimental.pallas import tpu as pltpu
from jax.experimental.pallas import tpu_sc as plsc

_FIELD = 40000
_NF = 26
_NE = 22
_B = 16384
_D = 16
_NW = 32
_PT = _B // _NW
_C = 64
_G = _PT // _C
_ER = _C * _NE // 128
_LR = _C * _NF // 128
_OFFS = np.arange(_NF, dtype=np.int32) * _FIELD

_mesh = plsc.VectorSubcoreMesh(core_axis_name="c", subcore_axis_name="s")


@functools.partial(
    pl.kernel,
    out_type=jax.ShapeDtypeStruct((_B,), jnp.float32),
    mesh=_mesh,
    compiler_params=pltpu.CompilerParams(
        needs_layout_passes=False, use_tc_tiling_on_sc=False),
    scratch_types=[
        pltpu.VMEM((_C * _NE,), jnp.int32),
        pltpu.VMEM((_C * _NF,), jnp.int32),
        pltpu.VMEM((_C * _NE, _D), jnp.float32),
        pltpu.VMEM((_C * _NF + 16,), jnp.float32),
        pltpu.VMEM((_D * 16,), jnp.float32),
        pltpu.VMEM((_C,), jnp.float32),
        pltpu.VMEM((16,), jnp.float32),
        pltpu.SemaphoreType.DMA,
    ],
)
def _fm_sc(eidx_h, lidx_h, emb_h, lin_h, bias_h, out_h,
           eidx, lidx, embbuf, linbuf, tbuf, zbuf, biasv, sem):
    wid = lax.axis_index("s") * 2 + lax.axis_index("c")
    pltpu.sync_copy(bias_h, biasv)
    lanes = lax.iota(jnp.int32, 16)
    mask10 = lanes < 10

    def chunk(g, carry):
        ch = wid * _G + g
        pltpu.sync_copy(eidx_h.at[pl.ds(ch * (_C * _NE), _C * _NE)], eidx)
        pltpu.sync_copy(lidx_h.at[pl.ds(ch * (_C * _NF), _C * _NF)], lidx)
        cps = []
        for j in range(_ER):
            cps.append(pltpu.async_copy(
                emb_h.at[eidx.at[pl.ds(j * 128, 128)]],
                embbuf.at[pl.ds(j * 128, 128)], sem))
        for j in range(_LR):
            cps.append(pltpu.async_copy(
                lin_h.at[lidx.at[pl.ds(j * 128, 128)]],
                linbuf.at[pl.ds(j * 128, 128)], sem))
        for c in cps:
            c.wait()
        bv = biasv[...]

        def per_grp(grp, c2):
            def per_b(j, c3):
                b = grp * 16 + j
                eb = b * _NE
                E = [embbuf[eb + i, :] for i in range(_NE)]
                s = E[4]
                for i in range(5, 11):
                    s = s + E[i]
                acc = E[3] * s
                t = s + E[11] + E[12]
                t = t + E[3]
                acc = acc + E[2] * t
                t = t + E[2]
                acc = acc + E[1] * t
                t = t + E[1]
                acc = acc + E[0] * t
                for i in range(20):
                    acc = acc + E[i] * (E[i + 1] * E[i + 2])
                lb = b * _NF
                v1 = linbuf[pl.ds(lb, 16)]
                v2 = linbuf[pl.ds(lb + 16, 16)]
                v2 = jnp.where(mask10, v2, jnp.float32(0.0))
                acc = acc + v1 + v2 + bv
                plsc.store_scatter(tbuf, [lanes * 16 + j], acc)
                return c3

            lax.fori_loop(0, 16, per_b, 0)
            z = tbuf[pl.ds(0, 16)]
            for d in range(1, _D):
                z = z + tbuf[pl.ds(d * 16, 16)]
            zbuf[pl.ds(grp * 16, 16)] = 1.0 / (1.0 + jnp.exp(-z))
            return c2

        lax.fori_loop(0, _C // 16, per_grp, 0)
        pltpu.sync_copy(zbuf, out_h.at[pl.ds(ch * _C, _C)])
        return carry

    lax.fori_loop(0, _G, chunk, 0)


def kernel(x, emb_table, lin_table, lin_bias, genotype_2nd, genotype_3rd):
    del genotype_2nd, genotype_3rd
    xo = x + jnp.asarray(_OFFS)[None, :]
    eidx = xo[:, :_NE].reshape(-1)
    lidx = xo.reshape(-1)
    bias16 = jnp.pad(lin_bias.astype(jnp.float32), (0, 15))
    return _fm_sc(eidx, lidx, emb_table[:_NE * _FIELD], lin_table.reshape(-1),
                  bias16)
